# baseline (speedup 1.0000x reference)
import functools

import jax
import jax.numpy as jnp
from jax import lax
from jax.experimental import pallas as pl
from jax.experimental.pallas import tpu as pltpu
from jax.experimental.pallas import tpu_sc as plsc

_EPS = 1e-12

_NC = 2
_NS = 16
_NW = _NC * _NS
_STREAM = 128


def _sc_gather_kernel(n_streams, idx_hbm, table_hbm, out_hbm, idx_v,
                      *slots):
    bufs = slots[:6]
    gsem = slots[6:]
    wid = lax.axis_index("s") * _NC + lax.axis_index("c")
    row0 = wid * n_streams
    pltpu.sync_copy(idx_hbm.at[pl.ds(row0, n_streams)], idx_v)

    def fire(j, k):
        pltpu.async_copy(table_hbm.at[idx_v.at[j]], bufs[k], gsem[k])

    def fire_if(j, k):
        @pl.when(j < n_streams)
        def _():
            fire(j, k)

    def drain_write(j, k):
        pltpu.make_async_copy(table_hbm.at[idx_v.at[0]], bufs[k],
                              gsem[k]).wait()
        pltpu.sync_copy(bufs[k], out_hbm.at[pl.ds((row0 + j) * _STREAM,
                                                  _STREAM)])

    fire(0, 0)
    fire(1, 1)
    fire(2, 2)

    def sextet(q, carry):
        j = 6 * q
        for k in range(6):
            fire_if(j + k + 3, (k + 3) % 6)
            drain_write(j + k, k)
        return carry

    n_full = n_streams // 6
    lax.fori_loop(0, n_full, sextet, 0, unroll=False)
    for t in range(n_full * 6, n_streams):
        drain_write(t, t % 6)


def _sc_gather(idx2, table):
    n_rows = idx2.shape[0] * idx2.shape[1]
    d = table.shape[1]
    n_streams = n_rows // (_NW * _STREAM)
    mesh = plsc.VectorSubcoreMesh(core_axis_name="c", subcore_axis_name="s")
    kern = pl.kernel(
        functools.partial(_sc_gather_kernel, n_streams),
        out_type=jax.ShapeDtypeStruct((n_rows, d), jnp.float32),
        mesh=mesh,
        scratch_types=(
            [pltpu.VMEM((n_streams, _STREAM), jnp.int32)]
            + [pltpu.VMEM((_STREAM, d), jnp.float32)] * 6
            + [pltpu.SemaphoreType.DMA] * 6
        ),
        compiler_params=pltpu.CompilerParams(use_tc_tiling_on_sc=True),
    )
    return kern(idx2, table)


def _tc_dense_kernel(embt_ref, wt_ref, b_ref, out_ref):
    x = embt_ref[...]
    s = jnp.sum(x * x, axis=0, keepdims=True)
    x = x * lax.rsqrt(jnp.maximum(s, _EPS * _EPS))
    x = jnp.tanh(x)
    h = lax.dot_general(x, wt_ref[...], (((0,), (0,)), ((), ())),
                        preferred_element_type=jnp.float32)
    h = h + b_ref[...]
    hs = jnp.sum(h * h, axis=1, keepdims=True)
    h = h * lax.rsqrt(jnp.maximum(hs, _EPS * _EPS))
    out_ref[...] = jnp.maximum(h, 0.0)


def _tc_dense(embt, wt, b2, block_n):
    d, n_rows = embt.shape
    out_dim = wt.shape[1]
    grid = (pl.cdiv(n_rows, block_n),)
    return pl.pallas_call(
        _tc_dense_kernel,
        grid=grid,
        in_specs=[
            pl.BlockSpec((d, block_n), lambda i: (0, i)),
            pl.BlockSpec((d, out_dim), lambda i: (0, 0)),
            pl.BlockSpec((1, out_dim), lambda i: (0, 0)),
        ],
        out_specs=pl.BlockSpec((block_n, out_dim), lambda i: (i, 0)),
        out_shape=jax.ShapeDtypeStruct((n_rows, out_dim), jnp.float32),
        compiler_params=pltpu.CompilerParams(
            dimension_semantics=("arbitrary",)),
    )(embt, wt, b2)


@jax.jit
def kernel(indices, table, W, b):
    n = indices.shape[0]
    out_dim = W.shape[0]
    b2 = b.reshape(1, out_dim)
    t4 = _tc_dense(table.T, W.T, b2, block_n=8192)
    idx2 = indices.astype(jnp.int32).reshape(n // _STREAM, _STREAM)
    return _sc_gather(idx2, t4)


# speedup vs baseline: 1.0028x; 1.0028x over previous
import functools

import jax
import jax.numpy as jnp
from jax import lax
from jax.experimental import pallas as pl
from jax.experimental.pallas import tpu as pltpu
from jax.experimental.pallas import tpu_sc as plsc

_EPS = 1e-12

_NC = 2
_NS = 16
_NW = _NC * _NS
_STREAM = 128


def _sc_gather_kernel(n_streams, idx_hbm, table_hbm, out_hbm, idx_v,
                      r0, r1, r2, r3, g0, g1, g2, g3):
    wid = lax.axis_index("s") * _NC + lax.axis_index("c")
    row0 = wid * n_streams
    pltpu.sync_copy(idx_hbm.at[pl.ds(row0, n_streams)], idx_v)

    def fire(j, buf, sem):
        pltpu.async_copy(table_hbm.at[idx_v.at[j]], buf, sem)

    def fire_if(j, buf, sem):
        @pl.when(j < n_streams)
        def _():
            fire(j, buf, sem)

    def drain_write(j, buf, sem):
        pltpu.make_async_copy(table_hbm.at[idx_v.at[0]], buf, sem).wait()
        pltpu.sync_copy(buf, out_hbm.at[pl.ds((row0 + j) * _STREAM,
                                              _STREAM)])

    fire(0, r0, g0)
    fire(1, r1, g1)

    def quad(q, carry):
        j = 4 * q
        fire(j + 2, r2, g2)
        drain_write(j, r0, g0)
        fire(j + 3, r3, g3)
        drain_write(j + 1, r1, g1)
        fire_if(j + 4, r0, g0)
        drain_write(j + 2, r2, g2)
        fire_if(j + 5, r1, g1)
        drain_write(j + 3, r3, g3)
        return carry

    lax.fori_loop(0, n_streams // 4, quad, 0, unroll=False)


def _sc_gather(idx2, table):
    n_rows = idx2.shape[0] * idx2.shape[1]
    d = table.shape[1]
    n_streams = n_rows // (_NW * _STREAM)
    mesh = plsc.VectorSubcoreMesh(core_axis_name="c", subcore_axis_name="s")
    kern = pl.kernel(
        functools.partial(_sc_gather_kernel, n_streams),
        out_type=jax.ShapeDtypeStruct((n_rows, d), jnp.float32),
        mesh=mesh,
        scratch_types=(
            [pltpu.VMEM((n_streams, _STREAM), jnp.int32)]
            + [pltpu.VMEM((_STREAM, d), jnp.float32)] * 4
            + [pltpu.SemaphoreType.DMA] * 4
        ),
        compiler_params=pltpu.CompilerParams(use_tc_tiling_on_sc=True),
    )
    return kern(idx2, table)


def _tc_dense_kernel(embt_ref, wt_ref, b_ref, out_ref):
    x = embt_ref[...]
    s = jnp.sum(x * x, axis=0, keepdims=True)
    x = x * lax.rsqrt(jnp.maximum(s, _EPS * _EPS))
    x = jnp.tanh(x)
    h = lax.dot_general(x, wt_ref[...], (((0,), (0,)), ((), ())),
                        preferred_element_type=jnp.float32)
    h = h + b_ref[...]
    hs = jnp.sum(h * h, axis=1, keepdims=True)
    h = h * lax.rsqrt(jnp.maximum(hs, _EPS * _EPS))
    out_ref[...] = jnp.maximum(h, 0.0)


def _tc_dense(embt, wt, b2, block_n):
    d, n_rows = embt.shape
    out_dim = wt.shape[1]
    grid = (pl.cdiv(n_rows, block_n),)
    return pl.pallas_call(
        _tc_dense_kernel,
        grid=grid,
        in_specs=[
            pl.BlockSpec((d, block_n), lambda i: (0, i)),
            pl.BlockSpec((d, out_dim), lambda i: (0, 0)),
            pl.BlockSpec((1, out_dim), lambda i: (0, 0)),
        ],
        out_specs=pl.BlockSpec((block_n, out_dim), lambda i: (i, 0)),
        out_shape=jax.ShapeDtypeStruct((n_rows, out_dim), jnp.float32),
        compiler_params=pltpu.CompilerParams(
            dimension_semantics=("parallel",)),
    )(embt, wt, b2)


@jax.jit
def kernel(indices, table, W, b):
    n = indices.shape[0]
    out_dim = W.shape[0]
    b2 = b.reshape(1, out_dim)
    t4 = _tc_dense(table.T, W.T, b2, block_n=8192)
    idx2 = indices.astype(jnp.int32).reshape(n // _STREAM, _STREAM)
    return _sc_gather(idx2, t4)
